# Initial kernel scaffold; baseline (speedup 1.0000x reference)
import jax
import jax.numpy as jnp
from jax.experimental import pallas as pl


def kernel(inputs, edge_index, edge_weight, W1, b1, W2, b2, Wc1, bc1, Wc2, bc2, Wf, bf):
    raise NotImplementedError("write your pallas kernel here")


# speedup vs baseline: 4.2900x; 4.2900x over previous
import functools

import jax
import jax.numpy as jnp
from jax import lax
from jax.experimental import pallas as pl
from jax.experimental.pallas import tpu as pltpu
from jax.experimental.pallas import tpu_sc as plsc

N = 10000
E = 320000
Q = 9
HID = 64
T_OUT = 12
DT = 0.1

NC = 2
NS = 16
NW = NC * NS
EW = E // NW
CH = 80
NCH = EW // CH
GR = CH // 16
NP = 10240
RP = NP // NS


def _f32(shape):
    return jax.ShapeDtypeStruct(shape, jnp.float32)


def _mesh():
    return plsc.VectorSubcoreMesh(
        core_axis_name="c", subcore_axis_name="s",
        num_cores=NC, num_subcores=NS)


def _sc_deg_body(dh, out, d_v, ones_v, z_v, acc):
    cid = lax.axis_index("c")
    sid = lax.axis_index("s")
    wid = sid * NC + cid
    row0 = sid * RP
    lanes = lax.iota(jnp.int32, 16)
    zvec = jnp.zeros((16,), jnp.float32)
    evec = jnp.where(lanes == 0, 1.0, 0.0).astype(jnp.float32)

    def fill(i, _):
        ones_v[i] = evec
        return 0

    lax.fori_loop(0, CH, fill, 0)

    def zrow(i, _):
        z_v[i] = zvec
        return 0

    lax.fori_loop(0, RP, zrow, 0)
    pltpu.sync_copy(z_v, acc.at[pl.ds(row0, RP)])
    plsc.subcore_barrier()

    def chunk(ch, _):
        base = wid * EW + ch * CH
        pltpu.sync_copy(dh.at[pl.ds(base, CH)], d_v)
        pltpu.sync_copy(ones_v, acc.at[d_v], add=True)
        return 0

    lax.fori_loop(0, NCH, chunk, 0)
    plsc.subcore_barrier()
    pltpu.sync_copy(acc.at[pl.ds(row0, RP)], out.at[cid, pl.ds(row0, RP)])


def _make_hop_body(D):
    def _hop(th, sh, dh, out, s_v, d_v, rows_v, z_v, acc):
        cid = lax.axis_index("c")
        sid = lax.axis_index("s")
        wid = sid * NC + cid
        row0 = sid * RP
        zvec = jnp.zeros((16,), jnp.float32)

        def zrow(i, _):
            for b in range(D // 16):
                z_v[i, pl.ds(b * 16, 16)] = zvec
            return 0

        lax.fori_loop(0, RP, zrow, 0)
        pltpu.sync_copy(z_v, acc.at[pl.ds(row0, RP)])
        plsc.subcore_barrier()

        def chunk(ch, _):
            base = wid * EW + ch * CH
            pltpu.sync_copy(sh.at[pl.ds(base, CH)], s_v)
            pltpu.sync_copy(dh.at[pl.ds(base, CH)], d_v)
            pltpu.sync_copy(th.at[s_v], rows_v)
            pltpu.sync_copy(rows_v, acc.at[d_v], add=True)
            return 0

        lax.fori_loop(0, NCH, chunk, 0)
        plsc.subcore_barrier()
        pltpu.sync_copy(acc.at[pl.ds(row0, RP)], out.at[cid, pl.ds(row0, RP)])

    return _hop


def _sc_edge_body(fh, sh, dh, wh, wfh, bfh, pin, pout,
                  f_v, s_v, d_v, w_v, wf_v, bf_v, pb_v, z_v, acc_i, acc_o):
    cid = lax.axis_index("c")
    sid = lax.axis_index("s")
    wid = sid * NC + cid
    row0 = sid * RP
    pltpu.sync_copy(fh, f_v)
    pltpu.sync_copy(wfh, wf_v)
    pltpu.sync_copy(bfh, bf_v)
    zvec = jnp.zeros((16,), jnp.float32)

    def zrow(i, _):
        z_v[i] = zvec
        return 0

    lax.fori_loop(0, RP, zrow, 0)
    pltpu.sync_copy(z_v, acc_i.at[pl.ds(row0, RP)])
    pltpu.sync_copy(z_v, acc_o.at[pl.ds(row0, RP)])

    def zp(i, _):
        pb_v[i] = zvec
        return 0

    lax.fori_loop(0, CH, zp, 0)
    plsc.subcore_barrier()
    lanes = lax.iota(jnp.int32, 16)

    def chunk(ch, _):
        base = wid * EW + ch * CH
        pltpu.sync_copy(sh.at[pl.ds(base, CH)], s_v)
        pltpu.sync_copy(dh.at[pl.ds(base, CH)], d_v)
        pltpu.sync_copy(wh.at[pl.ds(base, CH)], w_v)
        for g in range(GR):
            s16 = s_v[pl.ds(g * 16, 16)]
            d16 = d_v[pl.ds(g * 16, 16)]
            w16 = w_v[pl.ds(g * 16, 16)]
            sb = s16 * Q
            db = d16 * Q
            fs = [plsc.load_gather(f_v, [sb + q]) for q in range(Q)]
            fd = [plsc.load_gather(f_v, [db + q]) for q in range(Q)]
            rows = lanes + g * 16
            for j in range(Q):
                z = bf_v[j]
                for k in range(Q):
                    z = z + fs[k] * wf_v[k, j]
                for k in range(Q):
                    z = z + fd[k] * wf_v[Q + k, j]
                flow = w16 / (1.0 + jnp.exp(-z))
                pj = flow * fs[j]
                plsc.store_scatter(pb_v, [rows, lanes * 0 + j], pj)
        pltpu.sync_copy(pb_v, acc_i.at[d_v], add=True)
        pltpu.sync_copy(pb_v, acc_o.at[s_v], add=True)
        return 0

    lax.fori_loop(0, NCH, chunk, 0)
    plsc.subcore_barrier()
    pltpu.sync_copy(acc_i.at[pl.ds(row0, RP)], pin.at[cid, pl.ds(row0, RP)])
    pltpu.sync_copy(acc_o.at[pl.ds(row0, RP)], pout.at[cid, pl.ds(row0, RP)])


@functools.lru_cache(maxsize=1)
def _sc_kernels():
    mesh = _mesh()
    params = pltpu.CompilerParams(
        use_tc_tiling_on_sc=False, needs_layout_passes=False)
    sc_deg = pl.kernel(
        _sc_deg_body,
        out_type=_f32((NC, NP, 16)),
        mesh=mesh,
        compiler_params=params,
        scratch_types=[
            pltpu.VMEM((CH,), jnp.int32),
            pltpu.VMEM((CH, 16), jnp.float32),
            pltpu.VMEM((RP, 16), jnp.float32),
            pltpu.VMEM_SHARED((NP, 16), jnp.float32),
        ],
    )

    def hop(D):
        return pl.kernel(
            _make_hop_body(D),
            out_type=_f32((NC, NP, D)),
            mesh=mesh,
            compiler_params=params,
            scratch_types=[
                pltpu.VMEM((CH,), jnp.int32),
                pltpu.VMEM((CH,), jnp.int32),
                pltpu.VMEM((CH, D), jnp.float32),
                pltpu.VMEM((RP, D), jnp.float32),
                pltpu.VMEM_SHARED((NP, D), jnp.float32),
            ],
        )

    sc_edge = pl.kernel(
        _sc_edge_body,
        out_type=(_f32((NC, NP, 16)), _f32((NC, NP, 16))),
        mesh=mesh,
        compiler_params=params,
        scratch_types=[
            pltpu.VMEM((NP * Q,), jnp.float32),
            pltpu.VMEM((CH,), jnp.int32),
            pltpu.VMEM((CH,), jnp.int32),
            pltpu.VMEM((CH,), jnp.float32),
            pltpu.VMEM((2 * Q, Q, 16), jnp.float32),
            pltpu.VMEM((Q, 16), jnp.float32),
            pltpu.VMEM((CH, 16), jnp.float32),
            pltpu.VMEM((RP, 16), jnp.float32),
            pltpu.VMEM_SHARED((NP, 16), jnp.float32),
            pltpu.VMEM_SHARED((NP, 16), jnp.float32),
        ],
    )
    return sc_deg, hop(16), hop(64), sc_edge


def _tc_call(body, out_shapes, *args):
    return pl.pallas_call(body, out_shape=out_shapes)(*args)


def _norm_body(dp_ref, x_ref, norm_ref, t1_ref):
    deg = dp_ref[0, :, 0:1] + dp_ref[1, :, 0:1]
    norm = lax.rsqrt(jnp.maximum(deg, 1.0))
    norm_ref[...] = norm
    t1_ref[...] = x_ref[...] * norm


def _comb_body(hp_ref, n_ref, h_ref, t_ref):
    norm = n_ref[...]
    h = (hp_ref[0] + hp_ref[1]) * norm
    h_ref[...] = h
    t_ref[...] = h * norm


def _conv1_body(hp_ref, n_ref, x_ref, h1_ref, w_ref, b_ref, out_ref):
    h2 = (hp_ref[0] + hp_ref[1]) * n_ref[...]
    feats = jnp.concatenate(
        [x_ref[:, 0:2], h1_ref[:, 0:2], h2[:, 0:2]], axis=1)
    y = jnp.dot(feats, w_ref[...], preferred_element_type=jnp.float32)
    out_ref[...] = jnp.maximum(y + b_ref[...], 0.0)


def _premul_body(h_ref, n_ref, t_ref):
    t_ref[...] = h_ref[...] * n_ref[...]


def _conv2_body(gp_ref, n_ref, h_ref, g1_ref, w_ref, b_ref, out_ref):
    g2 = (gp_ref[0] + gp_ref[1]) * n_ref[...]
    feats = jnp.concatenate([h_ref[...], g1_ref[...], g2], axis=1)
    y = jnp.dot(feats, w_ref[...], preferred_element_type=jnp.float32) + b_ref[...]
    out_ref[...] = jnp.maximum(y, 0.0) + jnp.log1p(jnp.exp(-jnp.abs(y)))


def _update_body(f_ref, pi_ref, po_ref, wc1_ref, bc1_ref, wc2_ref, bc2_ref,
                 c2_ref, fn_ref, pred_ref, vl_ref):
    f = f_ref[...]
    infl = pi_ref[0, :, 0:Q] + pi_ref[1, :, 0:Q]
    outf = po_ref[0, :, 0:Q] + po_ref[1, :, 0:Q]
    hid = jnp.maximum(
        jnp.dot(f, wc1_ref[...], preferred_element_type=jnp.float32)
        + bc1_ref[...], 0.0)
    coll = jnp.dot(hid, wc2_ref[...], preferred_element_type=jnp.float32) \
        + bc2_ref[...]
    fn = jnp.maximum(f + DT * (infl - outf + coll), 0.0)
    fn_ref[...] = fn
    fnv = fn[0:N]
    dens = jnp.sum(fnv, axis=1, keepdims=True)
    moms = jnp.dot(fnv, c2_ref[...], preferred_element_type=jnp.float32)
    vel = moms[:, 0:1] / (dens + 1e-6)
    e2 = moms[:, 1:2] / (dens + 1e-6)
    pred_ref[...] = jnp.concatenate([dens, vel], axis=1)
    vl_ref[...] = jnp.sum(e2 - vel * vel).reshape(1, 1)


def kernel(inputs, edge_index, edge_weight, W1, b1, W2, b2,
           Wc1, bc1, Wc2, bc2, Wf, bf):
    _sc_deg, _sc_hop16, _sc_hop64, _sc_edge = _sc_kernels()
    src = edge_index[0]
    dst = edge_index[1]
    x = inputs[0, -1]
    xpad = jnp.pad(x, ((0, NP - N), (0, 14)))

    dp = _sc_deg(dst)
    norm, t1 = _tc_call(_norm_body, (_f32((NP, 1)), _f32((NP, 16))), dp, xpad)
    hp1 = _sc_hop16(t1, src, dst)
    h1, t2 = _tc_call(_comb_body, (_f32((NP, 16)), _f32((NP, 16))), hp1, norm)
    hp2 = _sc_hop16(t2, src, dst)
    h = _tc_call(_conv1_body, _f32((NP, HID)), hp2, norm, xpad, h1, W1, b1)
    t3 = _tc_call(_premul_body, _f32((NP, HID)), h, norm)
    gp1 = _sc_hop64(t3, src, dst)
    g1, t4 = _tc_call(_comb_body, (_f32((NP, HID)), _f32((NP, HID))), gp1, norm)
    gp2 = _sc_hop64(t4, src, dst)
    f = _tc_call(_conv2_body, _f32((NP, Q)), gp2, norm, h, g1, W2, b2)

    c = jnp.linspace(-1.0, 1.0, Q, dtype=jnp.float32)
    c2 = jnp.stack([c, c * c], axis=1)
    wfb = jnp.broadcast_to(Wf[:, :, None], (2 * Q, Q, 16))
    bfb = jnp.broadcast_to(bf[:, None], (Q, 16))

    def step(t, carry):
        f, preds, vl = carry
        pin, pout = _sc_edge(f.reshape(-1), src, dst, edge_weight, wfb, bfb)
        fn, pred, vls = _tc_call(
            _update_body, (_f32((NP, Q)), _f32((N, 2)), _f32((1, 1))),
            f, pin, pout, Wc1, bc1, Wc2, bc2, c2)
        preds = lax.dynamic_update_slice(preds, pred[None], (t, 0, 0))
        return fn, preds, vl + vls[0, 0]

    preds0 = jnp.zeros((T_OUT, N, 2), jnp.float32)
    f, preds, vl = lax.fori_loop(0, T_OUT, step, (f, preds0, jnp.float32(0.0)))
    return preds[None], vl / (N * T_OUT)


# speedup vs baseline: 5.1633x; 1.2036x over previous
import functools

import jax
import jax.numpy as jnp
from jax import lax
from jax.experimental import pallas as pl
from jax.experimental.pallas import tpu as pltpu
from jax.experimental.pallas import tpu_sc as plsc

N = 10000
E = 320000
Q = 9
HID = 64
T_OUT = 12
DT = 0.1

NC = 2
NS = 16
NW = NC * NS
EW = E // NW
CH = 80
NCH = EW // CH
GR = CH // 16
NP = 10240
RP = NP // NS


def _f32(shape):
    return jax.ShapeDtypeStruct(shape, jnp.float32)


def _mesh():
    return plsc.VectorSubcoreMesh(
        core_axis_name="c", subcore_axis_name="s",
        num_cores=NC, num_subcores=NS)


def _sc_deg_body(dh, out, d_v, ones_v, z_v, acc):
    cid = lax.axis_index("c")
    sid = lax.axis_index("s")
    wid = sid * NC + cid
    row0 = sid * RP
    lanes = lax.iota(jnp.int32, 16)
    zvec = jnp.zeros((16,), jnp.float32)
    evec = jnp.where(lanes == 0, 1.0, 0.0).astype(jnp.float32)

    def fill(i, _):
        ones_v[i] = evec
        return 0

    lax.fori_loop(0, CH, fill, 0)

    def zrow(i, _):
        z_v[i] = zvec
        return 0

    lax.fori_loop(0, RP, zrow, 0)
    pltpu.sync_copy(z_v, acc.at[pl.ds(row0, RP)])
    plsc.subcore_barrier()

    def chunk(ch, _):
        base = wid * EW + ch * CH
        pltpu.sync_copy(dh.at[pl.ds(base, CH)], d_v)
        pltpu.sync_copy(ones_v, acc.at[d_v], add=True)
        return 0

    lax.fori_loop(0, NCH, chunk, 0)
    plsc.subcore_barrier()
    pltpu.sync_copy(acc.at[pl.ds(row0, RP)], out.at[cid, pl.ds(row0, RP)])


def _make_hop_body(D):
    def _hop(th, sh, dh, out, s_v, d_v, rows_v, z_v, acc):
        cid = lax.axis_index("c")
        sid = lax.axis_index("s")
        wid = sid * NC + cid
        row0 = sid * RP
        zvec = jnp.zeros((16,), jnp.float32)

        def zrow(i, _):
            for b in range(D // 16):
                z_v[i, pl.ds(b * 16, 16)] = zvec
            return 0

        lax.fori_loop(0, RP, zrow, 0)
        pltpu.sync_copy(z_v, acc.at[pl.ds(row0, RP)])
        plsc.subcore_barrier()

        def chunk(ch, _):
            base = wid * EW + ch * CH
            pltpu.sync_copy(sh.at[pl.ds(base, CH)], s_v)
            pltpu.sync_copy(dh.at[pl.ds(base, CH)], d_v)
            pltpu.sync_copy(th.at[s_v], rows_v)
            pltpu.sync_copy(rows_v, acc.at[d_v], add=True)
            return 0

        lax.fori_loop(0, NCH, chunk, 0)
        plsc.subcore_barrier()
        pltpu.sync_copy(acc.at[pl.ds(row0, RP)], out.at[cid, pl.ds(row0, RP)])

    return _hop


def _sc_edge_body(fh, sh, dh, wh, wfh, bfh, pin, pout,
                  f_v, s_v, d_v, w_v, si_v, di_v, wf_v, bf_v, pb_v, z_v,
                  se0, se1, ss0, ss1, acc_i, acc_o):
    cid = lax.axis_index("c")
    sid = lax.axis_index("s")
    wid = sid * NC + cid
    row0 = sid * RP
    se = (se0, se1)
    ss = (ss0, ss1)
    pltpu.sync_copy(fh, f_v)
    pltpu.sync_copy(wfh, wf_v)
    pltpu.sync_copy(bfh, bf_v)
    zvec = jnp.zeros((16,), jnp.float32)

    def zrow(i, _):
        z_v[i] = zvec
        return 0

    lax.fori_loop(0, RP, zrow, 0)
    pltpu.sync_copy(z_v, acc_i.at[pl.ds(row0, RP)])
    pltpu.sync_copy(z_v, acc_o.at[pl.ds(row0, RP)])

    def zp(i, _):
        for b in range(2):
            pb_v[b, i] = zvec
        return 0

    lax.fori_loop(0, CH, zp, 0)
    plsc.subcore_barrier()
    lanes = lax.iota(jnp.int32, 16)

    def load(c, b):
        base = wid * EW + c * CH
        pltpu.async_copy(sh.at[pl.ds(base, CH)], s_v.at[b], se[b])
        pltpu.async_copy(dh.at[pl.ds(base, CH)], d_v.at[b], se[b])
        pltpu.async_copy(wh.at[pl.ds(base, CH)], w_v.at[b], se[b])

    def wait_load(b):
        pltpu.make_async_copy(sh.at[pl.ds(0, CH)], s_v.at[b], se[b]).wait()
        pltpu.make_async_copy(dh.at[pl.ds(0, CH)], d_v.at[b], se[b]).wait()
        pltpu.make_async_copy(wh.at[pl.ds(0, CH)], w_v.at[b], se[b]).wait()

    def scat(b):
        pltpu.async_copy(pb_v.at[b], acc_i.at[di_v.at[b]], ss[b], add=True)
        pltpu.async_copy(pb_v.at[b], acc_o.at[si_v.at[b]], ss[b], add=True)

    def wait_scat(b):
        pltpu.make_async_copy(pb_v.at[b], acc_i.at[di_v.at[b]], ss[b]).wait()
        pltpu.make_async_copy(pb_v.at[b], acc_o.at[si_v.at[b]], ss[b]).wait()

    def compute(b):
        for g in range(GR):
            si_v[b, pl.ds(g * 16, 16)] = s_v[b, pl.ds(g * 16, 16)]
            di_v[b, pl.ds(g * 16, 16)] = d_v[b, pl.ds(g * 16, 16)]
        for g in range(GR):
            s16 = si_v[b, pl.ds(g * 16, 16)]
            d16 = di_v[b, pl.ds(g * 16, 16)]
            w16 = w_v[b, pl.ds(g * 16, 16)]
            sb = s16 * Q
            db = d16 * Q
            fs = [plsc.load_gather(f_v, [sb + q]) for q in range(Q)]
            fd = [plsc.load_gather(f_v, [db + q]) for q in range(Q)]
            rows = lanes + g * 16
            for j in range(Q):
                z = bf_v[j]
                for k in range(Q):
                    z = z + fs[k] * wf_v[k, j]
                for k in range(Q):
                    z = z + fd[k] * wf_v[Q + k, j]
                flow = w16 / (1.0 + jnp.exp(-z))
                pj = flow * fs[j]
                plsc.store_scatter(pb_v.at[b], [rows, lanes * 0 + j], pj)

    load(0, 0)

    def pair(k, _):
        for half in range(2):
            c = 2 * k + half
            b = half
            wait_load(b)

            @pl.when(k >= 1)
            def _():
                wait_scat(b)

            load(c + 1, 1 - b)
            compute(b)
            scat(b)
        return 0

    lax.fori_loop(0, (NCH - 1) // 2, pair, 0)
    wait_load(0)
    wait_scat(0)
    compute(0)
    scat(0)
    wait_scat(0)
    wait_scat(1)
    plsc.subcore_barrier()
    pltpu.sync_copy(acc_i.at[pl.ds(row0, RP)], pin.at[cid, pl.ds(row0, RP)])
    pltpu.sync_copy(acc_o.at[pl.ds(row0, RP)], pout.at[cid, pl.ds(row0, RP)])


@functools.lru_cache(maxsize=1)
def _sc_kernels():
    mesh = _mesh()
    params = pltpu.CompilerParams(
        use_tc_tiling_on_sc=False, needs_layout_passes=False)
    sc_deg = pl.kernel(
        _sc_deg_body,
        out_type=_f32((NC, NP, 16)),
        mesh=mesh,
        compiler_params=params,
        scratch_types=[
            pltpu.VMEM((CH,), jnp.int32),
            pltpu.VMEM((CH, 16), jnp.float32),
            pltpu.VMEM((RP, 16), jnp.float32),
            pltpu.VMEM_SHARED((NP, 16), jnp.float32),
        ],
    )

    def hop(D):
        return pl.kernel(
            _make_hop_body(D),
            out_type=_f32((NC, NP, D)),
            mesh=mesh,
            compiler_params=params,
            scratch_types=[
                pltpu.VMEM((CH,), jnp.int32),
                pltpu.VMEM((CH,), jnp.int32),
                pltpu.VMEM((CH, D), jnp.float32),
                pltpu.VMEM((RP, D), jnp.float32),
                pltpu.VMEM_SHARED((NP, D), jnp.float32),
            ],
        )

    sc_edge = pl.kernel(
        _sc_edge_body,
        out_type=(_f32((NC, NP, 16)), _f32((NC, NP, 16))),
        mesh=mesh,
        compiler_params=params,
        scratch_types=[
            pltpu.VMEM((NP * Q,), jnp.float32),
            pltpu.VMEM((2, CH), jnp.int32),
            pltpu.VMEM((2, CH), jnp.int32),
            pltpu.VMEM((2, CH), jnp.float32),
            pltpu.VMEM((2, CH), jnp.int32),
            pltpu.VMEM((2, CH), jnp.int32),
            pltpu.VMEM((2 * Q, Q, 16), jnp.float32),
            pltpu.VMEM((Q, 16), jnp.float32),
            pltpu.VMEM((2, CH, 16), jnp.float32),
            pltpu.VMEM((RP, 16), jnp.float32),
            pltpu.SemaphoreType.DMA,
            pltpu.SemaphoreType.DMA,
            pltpu.SemaphoreType.DMA,
            pltpu.SemaphoreType.DMA,
            pltpu.VMEM_SHARED((NP, 16), jnp.float32),
            pltpu.VMEM_SHARED((NP, 16), jnp.float32),
        ],
    )
    return sc_deg, hop(16), hop(64), sc_edge


def _tc_call(body, out_shapes, *args):
    return pl.pallas_call(body, out_shape=out_shapes)(*args)


def _norm_body(dp_ref, x_ref, norm_ref, t1_ref):
    deg = dp_ref[0, :, 0:1] + dp_ref[1, :, 0:1]
    norm = lax.rsqrt(jnp.maximum(deg, 1.0))
    norm_ref[...] = norm
    t1_ref[...] = x_ref[...] * norm


def _comb_body(hp_ref, n_ref, h_ref, t_ref):
    norm = n_ref[...]
    h = (hp_ref[0] + hp_ref[1]) * norm
    h_ref[...] = h
    t_ref[...] = h * norm


def _conv1_body(hp_ref, n_ref, x_ref, h1_ref, w_ref, b_ref, out_ref):
    h2 = (hp_ref[0] + hp_ref[1]) * n_ref[...]
    feats = jnp.concatenate(
        [x_ref[:, 0:2], h1_ref[:, 0:2], h2[:, 0:2]], axis=1)
    y = jnp.dot(feats, w_ref[...], preferred_element_type=jnp.float32)
    out_ref[...] = jnp.maximum(y + b_ref[...], 0.0)


def _premul_body(h_ref, n_ref, t_ref):
    t_ref[...] = h_ref[...] * n_ref[...]


def _conv2_body(gp_ref, n_ref, h_ref, g1_ref, w_ref, b_ref, out_ref):
    g2 = (gp_ref[0] + gp_ref[1]) * n_ref[...]
    feats = jnp.concatenate([h_ref[...], g1_ref[...], g2], axis=1)
    y = jnp.dot(feats, w_ref[...], preferred_element_type=jnp.float32) + b_ref[...]
    out_ref[...] = jnp.maximum(y, 0.0) + jnp.log1p(jnp.exp(-jnp.abs(y)))


def _update_body(f_ref, pi_ref, po_ref, wc1_ref, bc1_ref, wc2_ref, bc2_ref,
                 c2_ref, fn_ref, pred_ref, vl_ref):
    f = f_ref[...]
    infl = pi_ref[0, :, 0:Q] + pi_ref[1, :, 0:Q]
    outf = po_ref[0, :, 0:Q] + po_ref[1, :, 0:Q]
    hid = jnp.maximum(
        jnp.dot(f, wc1_ref[...], preferred_element_type=jnp.float32)
        + bc1_ref[...], 0.0)
    coll = jnp.dot(hid, wc2_ref[...], preferred_element_type=jnp.float32) \
        + bc2_ref[...]
    fn = jnp.maximum(f + DT * (infl - outf + coll), 0.0)
    fn_ref[...] = fn
    fnv = fn[0:N]
    dens = jnp.sum(fnv, axis=1, keepdims=True)
    moms = jnp.dot(fnv, c2_ref[...], preferred_element_type=jnp.float32)
    vel = moms[:, 0:1] / (dens + 1e-6)
    e2 = moms[:, 1:2] / (dens + 1e-6)
    pred_ref[...] = jnp.concatenate([dens, vel], axis=1)
    vl_ref[...] = jnp.sum(e2 - vel * vel).reshape(1, 1)


def kernel(inputs, edge_index, edge_weight, W1, b1, W2, b2,
           Wc1, bc1, Wc2, bc2, Wf, bf):
    _sc_deg, _sc_hop16, _sc_hop64, _sc_edge = _sc_kernels()
    src = edge_index[0]
    dst = edge_index[1]
    x = inputs[0, -1]
    xpad = jnp.pad(x, ((0, NP - N), (0, 14)))

    dp = _sc_deg(dst)
    norm, t1 = _tc_call(_norm_body, (_f32((NP, 1)), _f32((NP, 16))), dp, xpad)
    hp1 = _sc_hop16(t1, src, dst)
    h1, t2 = _tc_call(_comb_body, (_f32((NP, 16)), _f32((NP, 16))), hp1, norm)
    hp2 = _sc_hop16(t2, src, dst)
    h = _tc_call(_conv1_body, _f32((NP, HID)), hp2, norm, xpad, h1, W1, b1)
    t3 = _tc_call(_premul_body, _f32((NP, HID)), h, norm)
    gp1 = _sc_hop64(t3, src, dst)
    g1, t4 = _tc_call(_comb_body, (_f32((NP, HID)), _f32((NP, HID))), gp1, norm)
    gp2 = _sc_hop64(t4, src, dst)
    f = _tc_call(_conv2_body, _f32((NP, Q)), gp2, norm, h, g1, W2, b2)

    c = jnp.linspace(-1.0, 1.0, Q, dtype=jnp.float32)
    c2 = jnp.stack([c, c * c], axis=1)
    wfb = jnp.broadcast_to(Wf[:, :, None], (2 * Q, Q, 16))
    bfb = jnp.broadcast_to(bf[:, None], (Q, 16))

    def step(t, carry):
        f, preds, vl = carry
        pin, pout = _sc_edge(f.reshape(-1), src, dst, edge_weight, wfb, bfb)
        fn, pred, vls = _tc_call(
            _update_body, (_f32((NP, Q)), _f32((N, 2)), _f32((1, 1))),
            f, pin, pout, Wc1, bc1, Wc2, bc2, c2)
        preds = lax.dynamic_update_slice(preds, pred[None], (t, 0, 0))
        return fn, preds, vl + vls[0, 0]

    preds0 = jnp.zeros((T_OUT, N, 2), jnp.float32)
    f, preds, vl = lax.fori_loop(0, T_OUT, step, (f, preds0, jnp.float32(0.0)))
    return preds[None], vl / (N * T_OUT)


# speedup vs baseline: 5.5748x; 1.0797x over previous
import functools

import jax
import jax.numpy as jnp
from jax import lax
from jax.experimental import pallas as pl
from jax.experimental.pallas import tpu as pltpu
from jax.experimental.pallas import tpu_sc as plsc

N = 10000
E = 320000
Q = 9
HID = 64
T_OUT = 12
DT = 0.1

NC = 2
NS = 16
NW = NC * NS
EW = E // NW
CH = 80
NCH = EW // CH
GR = CH // 16
NP = 10240
RP = NP // NS


def _f32(shape):
    return jax.ShapeDtypeStruct(shape, jnp.float32)


def _mesh():
    return plsc.VectorSubcoreMesh(
        core_axis_name="c", subcore_axis_name="s",
        num_cores=NC, num_subcores=NS)


def _sc_deg_body(dh, out, d_v, ones_v, z_v, acc):
    cid = lax.axis_index("c")
    sid = lax.axis_index("s")
    wid = sid * NC + cid
    row0 = sid * RP
    lanes = lax.iota(jnp.int32, 16)
    zvec = jnp.zeros((16,), jnp.float32)
    evec = jnp.where(lanes == 0, 1.0, 0.0).astype(jnp.float32)

    def fill(i, _):
        ones_v[i] = evec
        return 0

    lax.fori_loop(0, CH, fill, 0)

    def zrow(i, _):
        z_v[i] = zvec
        return 0

    lax.fori_loop(0, RP, zrow, 0)
    pltpu.sync_copy(z_v, acc.at[pl.ds(row0, RP)])
    plsc.subcore_barrier()

    def chunk(ch, _):
        base = wid * EW + ch * CH
        pltpu.sync_copy(dh.at[pl.ds(base, CH)], d_v)
        pltpu.sync_copy(ones_v, acc.at[d_v], add=True)
        return 0

    lax.fori_loop(0, NCH, chunk, 0)
    plsc.subcore_barrier()
    pltpu.sync_copy(acc.at[pl.ds(row0, RP)], out.at[cid, pl.ds(row0, RP)])


def _make_hop_body(D):
    def _hop(th, sh, dh, out, s_v, d_v, rows_v, z_v, acc):
        cid = lax.axis_index("c")
        sid = lax.axis_index("s")
        wid = sid * NC + cid
        row0 = sid * RP
        zvec = jnp.zeros((16,), jnp.float32)

        def zrow(i, _):
            for b in range(D // 16):
                z_v[i, pl.ds(b * 16, 16)] = zvec
            return 0

        lax.fori_loop(0, RP, zrow, 0)
        pltpu.sync_copy(z_v, acc.at[pl.ds(row0, RP)])
        plsc.subcore_barrier()

        def chunk(ch, _):
            base = wid * EW + ch * CH
            pltpu.sync_copy(sh.at[pl.ds(base, CH)], s_v)
            pltpu.sync_copy(dh.at[pl.ds(base, CH)], d_v)
            pltpu.sync_copy(th.at[s_v], rows_v)
            pltpu.sync_copy(rows_v, acc.at[d_v], add=True)
            return 0

        lax.fori_loop(0, NCH, chunk, 0)
        plsc.subcore_barrier()
        pltpu.sync_copy(acc.at[pl.ds(row0, RP)], out.at[cid, pl.ds(row0, RP)])

    return _hop


def _sc_edge_body(fuh, vh, sh, dh, wh, pin, pout,
                  s_v, d_v, si_v, di_v, w_v, fu_v, vv_v, pb_v, z_v,
                  si0, si1, sg0, sg1, ss0, ss1, acc_i, acc_o):
    cid = lax.axis_index("c")
    sid = lax.axis_index("s")
    wid = sid * NC + cid
    row0 = sid * RP
    sidx = (si0, si1)
    sgat = (sg0, sg1)
    ssc = (ss0, ss1)
    zvec = jnp.zeros((16,), jnp.float32)

    def zrow(i, _):
        z_v[i] = zvec
        return 0

    lax.fori_loop(0, RP, zrow, 0)
    pltpu.sync_copy(z_v, acc_i.at[pl.ds(row0, RP)])
    pltpu.sync_copy(z_v, acc_o.at[pl.ds(row0, RP)])
    plsc.subcore_barrier()

    def idx_load(c, b):
        base = wid * EW + c * CH
        pltpu.async_copy(sh.at[pl.ds(base, CH)], s_v.at[b], sidx[b])
        pltpu.async_copy(dh.at[pl.ds(base, CH)], d_v.at[b], sidx[b])
        pltpu.async_copy(wh.at[pl.ds(base, CH)], w_v.at[b], sidx[b])

    def wait_idx(b):
        pltpu.make_async_copy(sh.at[pl.ds(0, CH)], s_v.at[b], sidx[b]).wait()
        pltpu.make_async_copy(dh.at[pl.ds(0, CH)], d_v.at[b], sidx[b]).wait()
        pltpu.make_async_copy(wh.at[pl.ds(0, CH)], w_v.at[b], sidx[b]).wait()

    def gath(b):
        pltpu.async_copy(fuh.at[s_v.at[b]], fu_v.at[b], sgat[b])
        pltpu.async_copy(vh.at[d_v.at[b]], vv_v.at[b], sgat[b])

    def wait_gath(b):
        pltpu.make_async_copy(fuh.at[s_v.at[b]], fu_v.at[b], sgat[b]).wait()
        pltpu.make_async_copy(vh.at[d_v.at[b]], vv_v.at[b], sgat[b]).wait()

    def scat(b):
        pltpu.async_copy(pb_v.at[b], acc_i.at[di_v.at[b]], ssc[b], add=True)
        pltpu.async_copy(pb_v.at[b], acc_o.at[si_v.at[b]], ssc[b], add=True)

    def wait_scat(b):
        pltpu.make_async_copy(pb_v.at[b], acc_i.at[di_v.at[b]], ssc[b]).wait()
        pltpu.make_async_copy(pb_v.at[b], acc_o.at[si_v.at[b]], ssc[b]).wait()

    def copy_idx(b):
        for g in range(GR):
            si_v[b, pl.ds(g * 16, 16)] = s_v[b, pl.ds(g * 16, 16)]
            di_v[b, pl.ds(g * 16, 16)] = d_v[b, pl.ds(g * 16, 16)]

    def compute(b):
        for r in range(CH):
            fs = fu_v[b, r, pl.ds(0, 16)]
            u = fu_v[b, r, pl.ds(16, 16)]
            z = u + vv_v[b, r]
            pb_v[b, r] = w_v[b, r] * fs / (1.0 + jnp.exp(-z))

    idx_load(0, 0)
    idx_load(1, 1)
    wait_idx(0)
    gath(0)

    def pair(k, _):
        wait_idx(1)
        gath(1)
        wait_gath(0)

        @pl.when(k >= 1)
        def _():
            wait_scat(0)

        copy_idx(0)
        compute(0)
        idx_load(2 * k + 2, 0)
        scat(0)
        wait_idx(0)
        gath(0)
        wait_gath(1)

        @pl.when(k >= 1)
        def _():
            wait_scat(1)

        copy_idx(1)
        compute(1)

        @pl.when(k < (NCH - 3) // 2)
        def _():
            idx_load(2 * k + 3, 1)

        scat(1)
        return 0

    lax.fori_loop(0, (NCH - 1) // 2, pair, 0)
    wait_gath(0)
    wait_scat(0)
    copy_idx(0)
    compute(0)
    scat(0)
    wait_scat(0)
    wait_scat(1)
    plsc.subcore_barrier()
    pltpu.sync_copy(acc_i.at[pl.ds(row0, RP)], pin.at[cid, pl.ds(row0, RP)])
    pltpu.sync_copy(acc_o.at[pl.ds(row0, RP)], pout.at[cid, pl.ds(row0, RP)])


@functools.lru_cache(maxsize=1)
def _sc_kernels():
    mesh = _mesh()
    params = pltpu.CompilerParams(
        use_tc_tiling_on_sc=False, needs_layout_passes=False)
    sc_deg = pl.kernel(
        _sc_deg_body,
        out_type=_f32((NC, NP, 16)),
        mesh=mesh,
        compiler_params=params,
        scratch_types=[
            pltpu.VMEM((CH,), jnp.int32),
            pltpu.VMEM((CH, 16), jnp.float32),
            pltpu.VMEM((RP, 16), jnp.float32),
            pltpu.VMEM_SHARED((NP, 16), jnp.float32),
        ],
    )

    def hop(D):
        return pl.kernel(
            _make_hop_body(D),
            out_type=_f32((NC, NP, D)),
            mesh=mesh,
            compiler_params=params,
            scratch_types=[
                pltpu.VMEM((CH,), jnp.int32),
                pltpu.VMEM((CH,), jnp.int32),
                pltpu.VMEM((CH, D), jnp.float32),
                pltpu.VMEM((RP, D), jnp.float32),
                pltpu.VMEM_SHARED((NP, D), jnp.float32),
            ],
        )

    sc_edge = pl.kernel(
        _sc_edge_body,
        out_type=(_f32((NC, NP, 16)), _f32((NC, NP, 16))),
        mesh=mesh,
        compiler_params=params,
        scratch_types=[
            pltpu.VMEM((2, CH), jnp.int32),
            pltpu.VMEM((2, CH), jnp.int32),
            pltpu.VMEM((2, CH), jnp.int32),
            pltpu.VMEM((2, CH), jnp.int32),
            pltpu.VMEM((2, CH, 16), jnp.float32),
            pltpu.VMEM((2, CH, 32), jnp.float32),
            pltpu.VMEM((2, CH, 16), jnp.float32),
            pltpu.VMEM((2, CH, 16), jnp.float32),
            pltpu.VMEM((RP, 16), jnp.float32),
            pltpu.SemaphoreType.DMA,
            pltpu.SemaphoreType.DMA,
            pltpu.SemaphoreType.DMA,
            pltpu.SemaphoreType.DMA,
            pltpu.SemaphoreType.DMA,
            pltpu.SemaphoreType.DMA,
            pltpu.VMEM_SHARED((NP, 16), jnp.float32),
            pltpu.VMEM_SHARED((NP, 16), jnp.float32),
        ],
    )
    return sc_deg, hop(16), hop(64), sc_edge


def _tc_call(body, out_shapes, *args):
    return pl.pallas_call(body, out_shape=out_shapes)(*args)


def _norm_body(dp_ref, x_ref, norm_ref, t1_ref):
    deg = dp_ref[0, :, 0:1] + dp_ref[1, :, 0:1]
    norm = lax.rsqrt(jnp.maximum(deg, 1.0))
    norm_ref[...] = norm
    t1_ref[...] = x_ref[...] * norm


def _comb_body(hp_ref, n_ref, h_ref, t_ref):
    norm = n_ref[...]
    h = (hp_ref[0] + hp_ref[1]) * norm
    h_ref[...] = h
    t_ref[...] = h * norm


def _conv1_body(hp_ref, n_ref, x_ref, h1_ref, w_ref, b_ref, out_ref):
    h2 = (hp_ref[0] + hp_ref[1]) * n_ref[...]
    feats = jnp.concatenate(
        [x_ref[:, 0:2], h1_ref[:, 0:2], h2[:, 0:2]], axis=1)
    y = jnp.dot(feats, w_ref[...], preferred_element_type=jnp.float32)
    out_ref[...] = jnp.maximum(y + b_ref[...], 0.0)


def _premul_body(h_ref, n_ref, t_ref):
    t_ref[...] = h_ref[...] * n_ref[...]


def _pack_fuv(f, wf, bf, fu_ref, v_ref):
    u = jnp.dot(f, wf[0:Q], preferred_element_type=jnp.float32)
    vt = jnp.dot(f, wf[Q:2 * Q], preferred_element_type=jnp.float32) + bf
    z7 = jnp.zeros_like(f[:, 0:16 - Q])
    fu_ref[...] = jnp.concatenate([f, z7, u, z7], axis=1)
    v_ref[...] = jnp.concatenate([vt, z7], axis=1)


def _conv2_body(gp_ref, n_ref, h_ref, g1_ref, w_ref, b_ref, wf_ref, bf_ref,
                out_ref, fu_ref, v_ref):
    g2 = (gp_ref[0] + gp_ref[1]) * n_ref[...]
    feats = jnp.concatenate([h_ref[...], g1_ref[...], g2], axis=1)
    y = jnp.dot(feats, w_ref[...], preferred_element_type=jnp.float32) + b_ref[...]
    f = jnp.maximum(y, 0.0) + jnp.log1p(jnp.exp(-jnp.abs(y)))
    out_ref[...] = f
    _pack_fuv(f, wf_ref[...], bf_ref[...], fu_ref, v_ref)


def _update_body(f_ref, pi_ref, po_ref, wc1_ref, bc1_ref, wc2_ref, bc2_ref,
                 c2_ref, wf_ref, bf_ref, fn_ref, fu_ref, v_ref, pred_ref,
                 vl_ref):
    f = f_ref[...]
    infl = pi_ref[0, :, 0:Q] + pi_ref[1, :, 0:Q]
    outf = po_ref[0, :, 0:Q] + po_ref[1, :, 0:Q]
    hid = jnp.maximum(
        jnp.dot(f, wc1_ref[...], preferred_element_type=jnp.float32)
        + bc1_ref[...], 0.0)
    coll = jnp.dot(hid, wc2_ref[...], preferred_element_type=jnp.float32) \
        + bc2_ref[...]
    fn = jnp.maximum(f + DT * (infl - outf + coll), 0.0)
    fn_ref[...] = fn
    _pack_fuv(fn, wf_ref[...], bf_ref[...], fu_ref, v_ref)
    fnv = fn[0:N]
    dens = jnp.sum(fnv, axis=1, keepdims=True)
    moms = jnp.dot(fnv, c2_ref[...], preferred_element_type=jnp.float32)
    vel = moms[:, 0:1] / (dens + 1e-6)
    e2 = moms[:, 1:2] / (dens + 1e-6)
    pred_ref[...] = jnp.concatenate([dens, vel], axis=1)
    vl_ref[...] = jnp.sum(e2 - vel * vel).reshape(1, 1)


def kernel(inputs, edge_index, edge_weight, W1, b1, W2, b2,
           Wc1, bc1, Wc2, bc2, Wf, bf):
    _sc_deg, _sc_hop16, _sc_hop64, _sc_edge = _sc_kernels()
    src = edge_index[0]
    dst = edge_index[1]
    x = inputs[0, -1]
    xpad = jnp.pad(x, ((0, NP - N), (0, 14)))

    dp = _sc_deg(dst)
    norm, t1 = _tc_call(_norm_body, (_f32((NP, 1)), _f32((NP, 16))), dp, xpad)
    hp1 = _sc_hop16(t1, src, dst)
    h1, t2 = _tc_call(_comb_body, (_f32((NP, 16)), _f32((NP, 16))), hp1, norm)
    hp2 = _sc_hop16(t2, src, dst)
    h = _tc_call(_conv1_body, _f32((NP, HID)), hp2, norm, xpad, h1, W1, b1)
    t3 = _tc_call(_premul_body, _f32((NP, HID)), h, norm)
    gp1 = _sc_hop64(t3, src, dst)
    g1, t4 = _tc_call(_comb_body, (_f32((NP, HID)), _f32((NP, HID))), gp1, norm)
    gp2 = _sc_hop64(t4, src, dst)
    f, fu, v = _tc_call(
        _conv2_body, (_f32((NP, Q)), _f32((NP, 32)), _f32((NP, 16))),
        gp2, norm, h, g1, W2, b2, Wf, bf)

    c = jnp.linspace(-1.0, 1.0, Q, dtype=jnp.float32)
    c2 = jnp.stack([c, c * c], axis=1)
    wrow = jnp.broadcast_to(edge_weight[:, None], (E, 16))

    def step(t, carry):
        f, fu, v, preds, vl = carry
        pin, pout = _sc_edge(fu, v, src, dst, wrow)
        fn, fu2, v2, pred, vls = _tc_call(
            _update_body,
            (_f32((NP, Q)), _f32((NP, 32)), _f32((NP, 16)),
             _f32((N, 2)), _f32((1, 1))),
            f, pin, pout, Wc1, bc1, Wc2, bc2, c2, Wf, bf)
        preds = lax.dynamic_update_slice(preds, pred[None], (t, 0, 0))
        return fn, fu2, v2, preds, vl + vls[0, 0]

    preds0 = jnp.zeros((T_OUT, N, 2), jnp.float32)
    f, fu, v, preds, vl = lax.fori_loop(
        0, T_OUT, step, (f, fu, v, preds0, jnp.float32(0.0)))
    return preds[None], vl / (N * T_OUT)


# speedup vs baseline: 5.6322x; 1.0103x over previous
import functools

import jax
import jax.numpy as jnp
from jax import lax
from jax.experimental import pallas as pl
from jax.experimental.pallas import tpu as pltpu
from jax.experimental.pallas import tpu_sc as plsc

N = 10000
E = 320000
Q = 9
HID = 64
T_OUT = 12
DT = 0.1

NC = 2
NS = 16
NW = NC * NS
EW = E // NW
CH = 80
NCH = EW // CH
GR = CH // 16
NP = 10240
RP = NP // NS


def _f32(shape):
    return jax.ShapeDtypeStruct(shape, jnp.float32)


def _mesh():
    return plsc.VectorSubcoreMesh(
        core_axis_name="c", subcore_axis_name="s",
        num_cores=NC, num_subcores=NS)


def _sc_deg_body(dh, out, d_v, ones_v, z_v, acc):
    cid = lax.axis_index("c")
    sid = lax.axis_index("s")
    wid = sid * NC + cid
    row0 = sid * RP
    lanes = lax.iota(jnp.int32, 16)
    zvec = jnp.zeros((16,), jnp.float32)
    evec = jnp.where(lanes == 0, 1.0, 0.0).astype(jnp.float32)

    def fill(i, _):
        ones_v[i] = evec
        return 0

    lax.fori_loop(0, CH, fill, 0)

    def zrow(i, _):
        z_v[i] = zvec
        return 0

    lax.fori_loop(0, RP, zrow, 0)
    pltpu.sync_copy(z_v, acc.at[pl.ds(row0, RP)])
    plsc.subcore_barrier()

    def chunk(ch, _):
        base = wid * EW + ch * CH
        pltpu.sync_copy(dh.at[pl.ds(base, CH)], d_v)
        pltpu.sync_copy(ones_v, acc.at[d_v], add=True)
        return 0

    lax.fori_loop(0, NCH, chunk, 0)
    plsc.subcore_barrier()
    pltpu.sync_copy(acc.at[pl.ds(row0, RP)], out.at[cid, pl.ds(row0, RP)])


def _make_hop_body(D):
    def _hop(th, sh, dh, out, s_v, d_v, rows_v, z_v, acc):
        cid = lax.axis_index("c")
        sid = lax.axis_index("s")
        wid = sid * NC + cid
        row0 = sid * RP
        zvec = jnp.zeros((16,), jnp.float32)

        def zrow(i, _):
            for b in range(D // 16):
                z_v[i, pl.ds(b * 16, 16)] = zvec
            return 0

        lax.fori_loop(0, RP, zrow, 0)
        pltpu.sync_copy(z_v, acc.at[pl.ds(row0, RP)])
        plsc.subcore_barrier()

        def chunk(ch, _):
            base = wid * EW + ch * CH
            pltpu.sync_copy(sh.at[pl.ds(base, CH)], s_v)
            pltpu.sync_copy(dh.at[pl.ds(base, CH)], d_v)
            pltpu.sync_copy(th.at[s_v], rows_v)
            pltpu.sync_copy(rows_v, acc.at[d_v], add=True)
            return 0

        lax.fori_loop(0, NCH, chunk, 0)
        plsc.subcore_barrier()
        pltpu.sync_copy(acc.at[pl.ds(row0, RP)], out.at[cid, pl.ds(row0, RP)])

    return _hop


def _sc_edge_body(fuh, vh, sh, dh, wh, pin, pout,
                  s_v, d_v, si_v, di_v, w_v, fu_v, vv_v, pb_v, z_v,
                  si0, si1, sg0, sg1, ss0, ss1, acc_i, acc_o, fu_s, v_s):
    cid = lax.axis_index("c")
    sid = lax.axis_index("s")
    wid = sid * NC + cid
    row0 = sid * RP
    sidx = (si0, si1)
    sgat = (sg0, sg1)
    ssc = (ss0, ss1)
    zvec = jnp.zeros((16,), jnp.float32)

    pltpu.async_copy(fuh.at[pl.ds(row0, RP)], fu_s.at[pl.ds(row0, RP)], sg0)
    pltpu.async_copy(vh.at[pl.ds(row0, RP)], v_s.at[pl.ds(row0, RP)], sg1)

    def zrow(i, _):
        z_v[i] = zvec
        return 0

    lax.fori_loop(0, RP, zrow, 0)
    pltpu.sync_copy(z_v, acc_i.at[pl.ds(row0, RP)])
    pltpu.sync_copy(z_v, acc_o.at[pl.ds(row0, RP)])
    pltpu.make_async_copy(
        fuh.at[pl.ds(row0, RP)], fu_s.at[pl.ds(row0, RP)], sg0).wait()
    pltpu.make_async_copy(
        vh.at[pl.ds(row0, RP)], v_s.at[pl.ds(row0, RP)], sg1).wait()
    plsc.subcore_barrier()

    def idx_load(c, b):
        base = wid * EW + c * CH
        pltpu.async_copy(sh.at[pl.ds(base, CH)], s_v.at[b], sidx[b])
        pltpu.async_copy(dh.at[pl.ds(base, CH)], d_v.at[b], sidx[b])
        pltpu.async_copy(wh.at[pl.ds(base, CH)], w_v.at[b], sidx[b])

    def wait_idx(b):
        pltpu.make_async_copy(sh.at[pl.ds(0, CH)], s_v.at[b], sidx[b]).wait()
        pltpu.make_async_copy(dh.at[pl.ds(0, CH)], d_v.at[b], sidx[b]).wait()
        pltpu.make_async_copy(wh.at[pl.ds(0, CH)], w_v.at[b], sidx[b]).wait()

    def gath(b):
        pltpu.async_copy(fu_s.at[s_v.at[b]], fu_v.at[b], sgat[b])
        pltpu.async_copy(v_s.at[d_v.at[b]], vv_v.at[b], sgat[b])

    def wait_gath(b):
        pltpu.make_async_copy(fu_s.at[s_v.at[b]], fu_v.at[b], sgat[b]).wait()
        pltpu.make_async_copy(v_s.at[d_v.at[b]], vv_v.at[b], sgat[b]).wait()

    def scat(b):
        pltpu.async_copy(pb_v.at[b], acc_i.at[di_v.at[b]], ssc[b], add=True)
        pltpu.async_copy(pb_v.at[b], acc_o.at[si_v.at[b]], ssc[b], add=True)

    def wait_scat(b):
        pltpu.make_async_copy(pb_v.at[b], acc_i.at[di_v.at[b]], ssc[b]).wait()
        pltpu.make_async_copy(pb_v.at[b], acc_o.at[si_v.at[b]], ssc[b]).wait()

    def copy_idx(b):
        for g in range(GR):
            si_v[b, pl.ds(g * 16, 16)] = s_v[b, pl.ds(g * 16, 16)]
            di_v[b, pl.ds(g * 16, 16)] = d_v[b, pl.ds(g * 16, 16)]

    def compute(b):
        for r in range(CH):
            fs = fu_v[b, r, pl.ds(0, 16)]
            u = fu_v[b, r, pl.ds(16, 16)]
            z = u + vv_v[b, r]
            pb_v[b, r] = w_v[b, r] * fs / (1.0 + jnp.exp(-z))

    idx_load(0, 0)
    idx_load(1, 1)
    wait_idx(0)
    gath(0)

    def pair(k, _):
        wait_idx(1)
        gath(1)
        wait_gath(0)

        @pl.when(k >= 1)
        def _():
            wait_scat(0)

        copy_idx(0)
        compute(0)
        idx_load(2 * k + 2, 0)
        scat(0)
        wait_idx(0)
        gath(0)
        wait_gath(1)

        @pl.when(k >= 1)
        def _():
            wait_scat(1)

        copy_idx(1)
        compute(1)

        @pl.when(k < (NCH - 3) // 2)
        def _():
            idx_load(2 * k + 3, 1)

        scat(1)
        return 0

    lax.fori_loop(0, (NCH - 1) // 2, pair, 0)
    wait_gath(0)
    wait_scat(0)
    copy_idx(0)
    compute(0)
    scat(0)
    wait_scat(0)
    wait_scat(1)
    plsc.subcore_barrier()
    pltpu.sync_copy(acc_i.at[pl.ds(row0, RP)], pin.at[cid, pl.ds(row0, RP)])
    pltpu.sync_copy(acc_o.at[pl.ds(row0, RP)], pout.at[cid, pl.ds(row0, RP)])


@functools.lru_cache(maxsize=1)
def _sc_kernels():
    mesh = _mesh()
    params = pltpu.CompilerParams(
        use_tc_tiling_on_sc=False, needs_layout_passes=False)
    sc_deg = pl.kernel(
        _sc_deg_body,
        out_type=_f32((NC, NP, 16)),
        mesh=mesh,
        compiler_params=params,
        scratch_types=[
            pltpu.VMEM((CH,), jnp.int32),
            pltpu.VMEM((CH, 16), jnp.float32),
            pltpu.VMEM((RP, 16), jnp.float32),
            pltpu.VMEM_SHARED((NP, 16), jnp.float32),
        ],
    )

    def hop(D):
        return pl.kernel(
            _make_hop_body(D),
            out_type=_f32((NC, NP, D)),
            mesh=mesh,
            compiler_params=params,
            scratch_types=[
                pltpu.VMEM((CH,), jnp.int32),
                pltpu.VMEM((CH,), jnp.int32),
                pltpu.VMEM((CH, D), jnp.float32),
                pltpu.VMEM((RP, D), jnp.float32),
                pltpu.VMEM_SHARED((NP, D), jnp.float32),
            ],
        )

    sc_edge = pl.kernel(
        _sc_edge_body,
        out_type=(_f32((NC, NP, 16)), _f32((NC, NP, 16))),
        mesh=mesh,
        compiler_params=params,
        scratch_types=[
            pltpu.VMEM((2, CH), jnp.int32),
            pltpu.VMEM((2, CH), jnp.int32),
            pltpu.VMEM((2, CH), jnp.int32),
            pltpu.VMEM((2, CH), jnp.int32),
            pltpu.VMEM((2, CH, 16), jnp.float32),
            pltpu.VMEM((2, CH, 32), jnp.float32),
            pltpu.VMEM((2, CH, 16), jnp.float32),
            pltpu.VMEM((2, CH, 16), jnp.float32),
            pltpu.VMEM((RP, 16), jnp.float32),
            pltpu.SemaphoreType.DMA,
            pltpu.SemaphoreType.DMA,
            pltpu.SemaphoreType.DMA,
            pltpu.SemaphoreType.DMA,
            pltpu.SemaphoreType.DMA,
            pltpu.SemaphoreType.DMA,
            pltpu.VMEM_SHARED((NP, 16), jnp.float32),
            pltpu.VMEM_SHARED((NP, 16), jnp.float32),
            pltpu.VMEM_SHARED((NP, 32), jnp.float32),
            pltpu.VMEM_SHARED((NP, 16), jnp.float32),
        ],
    )
    return sc_deg, hop(16), hop(64), sc_edge


def _tc_call(body, out_shapes, *args):
    return pl.pallas_call(body, out_shape=out_shapes)(*args)


def _norm_body(dp_ref, x_ref, norm_ref, t1_ref):
    deg = dp_ref[0, :, 0:1] + dp_ref[1, :, 0:1]
    norm = lax.rsqrt(jnp.maximum(deg, 1.0))
    norm_ref[...] = norm
    t1_ref[...] = x_ref[...] * norm


def _comb_body(hp_ref, n_ref, h_ref, t_ref):
    norm = n_ref[...]
    h = (hp_ref[0] + hp_ref[1]) * norm
    h_ref[...] = h
    t_ref[...] = h * norm


def _conv1_body(hp_ref, n_ref, x_ref, h1_ref, w_ref, b_ref, out_ref):
    h2 = (hp_ref[0] + hp_ref[1]) * n_ref[...]
    feats = jnp.concatenate(
        [x_ref[:, 0:2], h1_ref[:, 0:2], h2[:, 0:2]], axis=1)
    y = jnp.dot(feats, w_ref[...], preferred_element_type=jnp.float32)
    out_ref[...] = jnp.maximum(y + b_ref[...], 0.0)


def _premul_body(h_ref, n_ref, t_ref):
    t_ref[...] = h_ref[...] * n_ref[...]


def _pack_fuv(f, wf, bf, fu_ref, v_ref):
    u = jnp.dot(f, wf[0:Q], preferred_element_type=jnp.float32)
    vt = jnp.dot(f, wf[Q:2 * Q], preferred_element_type=jnp.float32) + bf
    z7 = jnp.zeros_like(f[:, 0:16 - Q])
    fu_ref[...] = jnp.concatenate([f, z7, u, z7], axis=1)
    v_ref[...] = jnp.concatenate([vt, z7], axis=1)


def _conv2_body(gp_ref, n_ref, h_ref, g1_ref, w_ref, b_ref, wf_ref, bf_ref,
                out_ref, fu_ref, v_ref):
    g2 = (gp_ref[0] + gp_ref[1]) * n_ref[...]
    feats = jnp.concatenate([h_ref[...], g1_ref[...], g2], axis=1)
    y = jnp.dot(feats, w_ref[...], preferred_element_type=jnp.float32) + b_ref[...]
    f = jnp.maximum(y, 0.0) + jnp.log1p(jnp.exp(-jnp.abs(y)))
    out_ref[...] = f
    _pack_fuv(f, wf_ref[...], bf_ref[...], fu_ref, v_ref)


def _update_body(f_ref, pi_ref, po_ref, wc1_ref, bc1_ref, wc2_ref, bc2_ref,
                 c2_ref, wf_ref, bf_ref, fn_ref, fu_ref, v_ref, pred_ref,
                 vl_ref):
    f = f_ref[...]
    infl = pi_ref[0, :, 0:Q] + pi_ref[1, :, 0:Q]
    outf = po_ref[0, :, 0:Q] + po_ref[1, :, 0:Q]
    hid = jnp.maximum(
        jnp.dot(f, wc1_ref[...], preferred_element_type=jnp.float32)
        + bc1_ref[...], 0.0)
    coll = jnp.dot(hid, wc2_ref[...], preferred_element_type=jnp.float32) \
        + bc2_ref[...]
    fn = jnp.maximum(f + DT * (infl - outf + coll), 0.0)
    fn_ref[...] = fn
    _pack_fuv(fn, wf_ref[...], bf_ref[...], fu_ref, v_ref)
    fnv = fn[0:N]
    dens = jnp.sum(fnv, axis=1, keepdims=True)
    moms = jnp.dot(fnv, c2_ref[...], preferred_element_type=jnp.float32)
    vel = moms[:, 0:1] / (dens + 1e-6)
    e2 = moms[:, 1:2] / (dens + 1e-6)
    pred_ref[...] = jnp.concatenate([dens, vel], axis=1)
    vl_ref[...] = jnp.sum(e2 - vel * vel).reshape(1, 1)


def kernel(inputs, edge_index, edge_weight, W1, b1, W2, b2,
           Wc1, bc1, Wc2, bc2, Wf, bf):
    _sc_deg, _sc_hop16, _sc_hop64, _sc_edge = _sc_kernels()
    src = edge_index[0]
    dst = edge_index[1]
    x = inputs[0, -1]
    xpad = jnp.pad(x, ((0, NP - N), (0, 14)))

    dp = _sc_deg(dst)
    norm, t1 = _tc_call(_norm_body, (_f32((NP, 1)), _f32((NP, 16))), dp, xpad)
    hp1 = _sc_hop16(t1, src, dst)
    h1, t2 = _tc_call(_comb_body, (_f32((NP, 16)), _f32((NP, 16))), hp1, norm)
    hp2 = _sc_hop16(t2, src, dst)
    h = _tc_call(_conv1_body, _f32((NP, HID)), hp2, norm, xpad, h1, W1, b1)
    t3 = _tc_call(_premul_body, _f32((NP, HID)), h, norm)
    gp1 = _sc_hop64(t3, src, dst)
    g1, t4 = _tc_call(_comb_body, (_f32((NP, HID)), _f32((NP, HID))), gp1, norm)
    gp2 = _sc_hop64(t4, src, dst)
    f, fu, v = _tc_call(
        _conv2_body, (_f32((NP, Q)), _f32((NP, 32)), _f32((NP, 16))),
        gp2, norm, h, g1, W2, b2, Wf, bf)

    c = jnp.linspace(-1.0, 1.0, Q, dtype=jnp.float32)
    c2 = jnp.stack([c, c * c], axis=1)
    wrow = jnp.broadcast_to(edge_weight[:, None], (E, 16))

    def step(t, carry):
        f, fu, v, preds, vl = carry
        pin, pout = _sc_edge(fu, v, src, dst, wrow)
        fn, fu2, v2, pred, vls = _tc_call(
            _update_body,
            (_f32((NP, Q)), _f32((NP, 32)), _f32((NP, 16)),
             _f32((N, 2)), _f32((1, 1))),
            f, pin, pout, Wc1, bc1, Wc2, bc2, c2, Wf, bf)
        preds = lax.dynamic_update_slice(preds, pred[None], (t, 0, 0))
        return fn, fu2, v2, preds, vl + vls[0, 0]

    preds0 = jnp.zeros((T_OUT, N, 2), jnp.float32)
    f, fu, v, preds, vl = lax.fori_loop(
        0, T_OUT, step, (f, fu, v, preds0, jnp.float32(0.0)))
    return preds[None], vl / (N * T_OUT)


# speedup vs baseline: 8.5945x; 1.5260x over previous
import functools

import jax
import jax.numpy as jnp
from jax import lax
from jax.experimental import pallas as pl
from jax.experimental.pallas import tpu as pltpu
from jax.experimental.pallas import tpu_sc as plsc

N = 10000
E = 320000
Q = 9
HID = 64
T_OUT = 12
DT = 0.1

NC = 2
NS = 16
NW = NC * NS
EW = E // NW
CH = 80
NCH = EW // CH
GR = CH // 16
NP = 10240
RP = NP // NS


def _f32(shape):
    return jax.ShapeDtypeStruct(shape, jnp.float32)


def _mesh():
    return plsc.VectorSubcoreMesh(
        core_axis_name="c", subcore_axis_name="s",
        num_cores=NC, num_subcores=NS)


def _sc_deg_body(dh, out, d_v, ones_v, z_v, acc):
    cid = lax.axis_index("c")
    sid = lax.axis_index("s")
    wid = sid * NC + cid
    row0 = sid * RP
    lanes = lax.iota(jnp.int32, 16)
    zvec = jnp.zeros((16,), jnp.float32)
    evec = jnp.where(lanes == 0, 1.0, 0.0).astype(jnp.float32)

    def fill(i, _):
        ones_v[i] = evec
        return 0

    lax.fori_loop(0, CH, fill, 0)

    def zrow(i, _):
        z_v[i] = zvec
        return 0

    lax.fori_loop(0, RP, zrow, 0)
    pltpu.sync_copy(z_v, acc.at[pl.ds(row0, RP)])
    plsc.subcore_barrier()

    def chunk(ch, _):
        base = wid * EW + ch * CH
        pltpu.sync_copy(dh.at[pl.ds(base, CH)], d_v)
        pltpu.sync_copy(ones_v, acc.at[d_v], add=True)
        return 0

    lax.fori_loop(0, NCH, chunk, 0)
    plsc.subcore_barrier()
    pltpu.sync_copy(acc.at[pl.ds(row0, RP)], out.at[cid, pl.ds(row0, RP)])


def _make_hop_body(D):
    def _hop(th, sh, dh, out, s_v, d_v, rows_v, z_v, acc):
        cid = lax.axis_index("c")
        sid = lax.axis_index("s")
        wid = sid * NC + cid
        row0 = sid * RP
        zvec = jnp.zeros((16,), jnp.float32)

        def zrow(i, _):
            for b in range(D // 16):
                z_v[i, pl.ds(b * 16, 16)] = zvec
            return 0

        lax.fori_loop(0, RP, zrow, 0)
        pltpu.sync_copy(z_v, acc.at[pl.ds(row0, RP)])
        plsc.subcore_barrier()

        def chunk(ch, _):
            base = wid * EW + ch * CH
            pltpu.sync_copy(sh.at[pl.ds(base, CH)], s_v)
            pltpu.sync_copy(dh.at[pl.ds(base, CH)], d_v)
            pltpu.sync_copy(th.at[s_v], rows_v)
            pltpu.sync_copy(rows_v, acc.at[d_v], add=True)
            return 0

        lax.fori_loop(0, NCH, chunk, 0)
        plsc.subcore_barrier()
        pltpu.sync_copy(acc.at[pl.ds(row0, RP)], out.at[cid, pl.ds(row0, RP)])

    return _hop


def _sc_edge_body(fuh, vh, sh, dh, wh, pin, pout,
                  s_v, d_v, si_v, di_v, w_v, fu_v, vv_v, pb_v, z_v,
                  si0, si1, sg0, sg1, ss0, ss1, acc_i, acc_o, fu_s, v_s):
    cid = lax.axis_index("c")
    sid = lax.axis_index("s")
    wid = sid * NC + cid
    row0 = sid * RP
    sidx = (si0, si1)
    sgat = (sg0, sg1)
    ssc = (ss0, ss1)
    zvec = jnp.zeros((16,), jnp.float32)

    pltpu.async_copy(fuh.at[pl.ds(row0, RP)], fu_s.at[pl.ds(row0, RP)], sg0)
    pltpu.async_copy(vh.at[pl.ds(row0, RP)], v_s.at[pl.ds(row0, RP)], sg1)

    def zrow(i, _):
        z_v[i] = zvec
        return 0

    lax.fori_loop(0, RP, zrow, 0)
    pltpu.sync_copy(z_v, acc_i.at[pl.ds(row0, RP)])
    pltpu.sync_copy(z_v, acc_o.at[pl.ds(row0, RP)])
    pltpu.make_async_copy(
        fuh.at[pl.ds(row0, RP)], fu_s.at[pl.ds(row0, RP)], sg0).wait()
    pltpu.make_async_copy(
        vh.at[pl.ds(row0, RP)], v_s.at[pl.ds(row0, RP)], sg1).wait()
    plsc.subcore_barrier()

    def idx_load(c, b):
        base = wid * EW + c * CH
        pltpu.async_copy(sh.at[pl.ds(base, CH)], s_v.at[b], sidx[b])
        pltpu.async_copy(dh.at[pl.ds(base, CH)], d_v.at[b], sidx[b])
        pltpu.async_copy(wh.at[pl.ds(base, CH)], w_v.at[b], sidx[b])

    def wait_idx(b):
        pltpu.make_async_copy(sh.at[pl.ds(0, CH)], s_v.at[b], sidx[b]).wait()
        pltpu.make_async_copy(dh.at[pl.ds(0, CH)], d_v.at[b], sidx[b]).wait()
        pltpu.make_async_copy(wh.at[pl.ds(0, CH)], w_v.at[b], sidx[b]).wait()

    def gath(b):
        pltpu.async_copy(fu_s.at[s_v.at[b]], fu_v.at[b], sgat[b])
        pltpu.async_copy(v_s.at[d_v.at[b]], vv_v.at[b], sgat[b])

    def wait_gath(b):
        pltpu.make_async_copy(fu_s.at[s_v.at[b]], fu_v.at[b], sgat[b]).wait()
        pltpu.make_async_copy(v_s.at[d_v.at[b]], vv_v.at[b], sgat[b]).wait()

    def scat(b):
        pltpu.async_copy(pb_v.at[b], acc_i.at[di_v.at[b]], ssc[b], add=True)
        pltpu.async_copy(pb_v.at[b], acc_o.at[si_v.at[b]], ssc[b], add=True)

    def wait_scat(b):
        pltpu.make_async_copy(pb_v.at[b], acc_i.at[di_v.at[b]], ssc[b]).wait()
        pltpu.make_async_copy(pb_v.at[b], acc_o.at[si_v.at[b]], ssc[b]).wait()

    def copy_idx(b):
        for g in range(GR):
            si_v[b, pl.ds(g * 16, 16)] = s_v[b, pl.ds(g * 16, 16)]
            di_v[b, pl.ds(g * 16, 16)] = d_v[b, pl.ds(g * 16, 16)]

    def compute(b):
        for r in range(CH):
            fs = fu_v[b, r, pl.ds(0, 16)]
            u = fu_v[b, r, pl.ds(16, 16)]
            z = u + vv_v[b, r]
            pb_v[b, r] = w_v[b, r] * fs / (1.0 + jnp.exp(-z))

    idx_load(0, 0)
    idx_load(1, 1)
    wait_idx(0)
    gath(0)

    def pair(k, _):
        wait_idx(1)
        gath(1)
        wait_gath(0)

        @pl.when(k >= 1)
        def _():
            wait_scat(0)

        copy_idx(0)
        compute(0)
        idx_load(2 * k + 2, 0)
        scat(0)
        wait_idx(0)
        gath(0)
        wait_gath(1)

        @pl.when(k >= 1)
        def _():
            wait_scat(1)

        copy_idx(1)
        compute(1)

        @pl.when(k < (NCH - 3) // 2)
        def _():
            idx_load(2 * k + 3, 1)

        scat(1)
        return 0

    lax.fori_loop(0, (NCH - 1) // 2, pair, 0)
    wait_gath(0)
    wait_scat(0)
    copy_idx(0)
    compute(0)
    scat(0)
    wait_scat(0)
    wait_scat(1)
    plsc.subcore_barrier()
    pltpu.sync_copy(acc_i.at[pl.ds(row0, RP)], pin.at[cid, pl.ds(row0, RP)])
    pltpu.sync_copy(acc_o.at[pl.ds(row0, RP)], pout.at[cid, pl.ds(row0, RP)])


@functools.lru_cache(maxsize=1)
def _sc_kernels():
    mesh = _mesh()
    params = pltpu.CompilerParams(
        use_tc_tiling_on_sc=False, needs_layout_passes=False)
    sc_deg = pl.kernel(
        _sc_deg_body,
        out_type=_f32((NC, NP, 16)),
        mesh=mesh,
        compiler_params=params,
        scratch_types=[
            pltpu.VMEM((CH,), jnp.int32),
            pltpu.VMEM((CH, 16), jnp.float32),
            pltpu.VMEM((RP, 16), jnp.float32),
            pltpu.VMEM_SHARED((NP, 16), jnp.float32),
        ],
    )

    def hop(D):
        return pl.kernel(
            _make_hop_body(D),
            out_type=_f32((NC, NP, D)),
            mesh=mesh,
            compiler_params=params,
            scratch_types=[
                pltpu.VMEM((CH,), jnp.int32),
                pltpu.VMEM((CH,), jnp.int32),
                pltpu.VMEM((CH, D), jnp.float32),
                pltpu.VMEM((RP, D), jnp.float32),
                pltpu.VMEM_SHARED((NP, D), jnp.float32),
            ],
        )

    sc_edge = pl.kernel(
        _sc_edge_body,
        out_type=(_f32((NC, NP, 16)), _f32((NC, NP, 16))),
        mesh=mesh,
        compiler_params=params,
        scratch_types=[
            pltpu.VMEM((2, CH), jnp.int32),
            pltpu.VMEM((2, CH), jnp.int32),
            pltpu.VMEM((2, CH), jnp.int32),
            pltpu.VMEM((2, CH), jnp.int32),
            pltpu.VMEM((2, CH, 16), jnp.float32),
            pltpu.VMEM((2, CH, 32), jnp.float32),
            pltpu.VMEM((2, CH, 16), jnp.float32),
            pltpu.VMEM((2, CH, 16), jnp.float32),
            pltpu.VMEM((RP, 16), jnp.float32),
            pltpu.SemaphoreType.DMA,
            pltpu.SemaphoreType.DMA,
            pltpu.SemaphoreType.DMA,
            pltpu.SemaphoreType.DMA,
            pltpu.SemaphoreType.DMA,
            pltpu.SemaphoreType.DMA,
            pltpu.VMEM_SHARED((NP, 16), jnp.float32),
            pltpu.VMEM_SHARED((NP, 16), jnp.float32),
            pltpu.VMEM_SHARED((NP, 32), jnp.float32),
            pltpu.VMEM_SHARED((NP, 16), jnp.float32),
        ],
    )
    return sc_deg, hop(16), hop(64), sc_edge


def _tc_call(body, out_shapes, *args):
    return pl.pallas_call(body, out_shape=out_shapes)(*args)


def _norm_body(dp_ref, x_ref, norm_ref, t1_ref):
    deg = dp_ref[0, :, 0:1] + dp_ref[1, :, 0:1]
    norm = lax.rsqrt(jnp.maximum(deg, 1.0))
    norm_ref[...] = norm
    t1_ref[...] = x_ref[...] * norm


def _comb_body(hp_ref, n_ref, h_ref, t_ref):
    norm = n_ref[...]
    h = (hp_ref[0] + hp_ref[1]) * norm
    h_ref[...] = h
    t_ref[...] = h * norm


def _conv1_body(hp_ref, n_ref, x_ref, h1_ref, w_ref, b_ref, out_ref):
    h2 = (hp_ref[0] + hp_ref[1]) * n_ref[...]
    feats = jnp.concatenate(
        [x_ref[:, 0:2], h1_ref[:, 0:2], h2[:, 0:2]], axis=1)
    y = jnp.dot(feats, w_ref[...], preferred_element_type=jnp.float32)
    out_ref[...] = jnp.maximum(y + b_ref[...], 0.0)


def _premul_body(h_ref, n_ref, t_ref):
    t_ref[...] = h_ref[...] * n_ref[...]


def _pack_fuv(f, wf, bf, fu_ref, v_ref):
    u = jnp.dot(f, wf[0:Q], preferred_element_type=jnp.float32)
    vt = jnp.dot(f, wf[Q:2 * Q], preferred_element_type=jnp.float32) + bf
    z7 = jnp.zeros_like(f[:, 0:16 - Q])
    fu_ref[...] = jnp.concatenate([f, z7, u, z7], axis=1)
    v_ref[...] = jnp.concatenate([vt, z7], axis=1)


def _conv2_body(gp_ref, n_ref, h_ref, g1_ref, w_ref, b_ref, wf_ref, bf_ref,
                out_ref, fu_ref, v_ref):
    g2 = (gp_ref[0] + gp_ref[1]) * n_ref[...]
    feats = jnp.concatenate([h_ref[...], g1_ref[...], g2], axis=1)
    y = jnp.dot(feats, w_ref[...], preferred_element_type=jnp.float32) + b_ref[...]
    f = jnp.maximum(y, 0.0) + jnp.log1p(jnp.exp(-jnp.abs(y)))
    out_ref[...] = f
    _pack_fuv(f, wf_ref[...], bf_ref[...], fu_ref, v_ref)


def _update_body(f_ref, pi_ref, po_ref, wc1_ref, bc1_ref, wc2_ref, bc2_ref,
                 c2_ref, wf_ref, bf_ref, fn_ref, fu_ref, v_ref, pred_ref,
                 vl_ref):
    f = f_ref[...]
    infl = pi_ref[0, :, 0:Q] + pi_ref[1, :, 0:Q]
    outf = po_ref[0, :, 0:Q] + po_ref[1, :, 0:Q]
    hid = jnp.maximum(
        jnp.dot(f, wc1_ref[...], preferred_element_type=jnp.float32)
        + bc1_ref[...], 0.0)
    coll = jnp.dot(hid, wc2_ref[...], preferred_element_type=jnp.float32) \
        + bc2_ref[...]
    fn = jnp.maximum(f + DT * (infl - outf + coll), 0.0)
    fn_ref[...] = fn
    _pack_fuv(fn, wf_ref[...], bf_ref[...], fu_ref, v_ref)
    fnv = fn[0:N]
    dens = jnp.sum(fnv, axis=1, keepdims=True)
    moms = jnp.dot(fnv, c2_ref[...], preferred_element_type=jnp.float32)
    vel = moms[:, 0:1] / (dens + 1e-6)
    e2 = moms[:, 1:2] / (dens + 1e-6)
    pred_ref[...] = jnp.concatenate([dens, vel], axis=1)
    vl_ref[...] = jnp.sum(e2 - vel * vel).reshape(1, 1)


def kernel(inputs, edge_index, edge_weight, W1, b1, W2, b2,
           Wc1, bc1, Wc2, bc2, Wf, bf):
    _sc_deg, _sc_hop16, _sc_hop64, _sc_edge = _sc_kernels()
    src = edge_index[0]
    dst = edge_index[1]
    x = inputs[0, -1]
    xpad = jnp.pad(x, ((0, NP - N), (0, 14)))

    dp = _sc_deg(dst)
    norm, t1 = _tc_call(_norm_body, (_f32((NP, 1)), _f32((NP, 16))), dp, xpad)
    hp1 = _sc_hop16(t1, src, dst)
    h1, t2 = _tc_call(_comb_body, (_f32((NP, 16)), _f32((NP, 16))), hp1, norm)
    hp2 = _sc_hop16(t2, src, dst)
    h = _tc_call(_conv1_body, _f32((NP, HID)), hp2, norm, xpad, h1, W1, b1)
    t3 = _tc_call(_premul_body, _f32((NP, HID)), h, norm)
    gp1 = _sc_hop64(t3, src, dst)
    g1, t4 = _tc_call(_comb_body, (_f32((NP, HID)), _f32((NP, HID))), gp1, norm)
    gp2 = _sc_hop64(t4, src, dst)
    f, fu, v = _tc_call(
        _conv2_body, (_f32((NP, Q)), _f32((NP, 32)), _f32((NP, 16))),
        gp2, norm, h, g1, W2, b2, Wf, bf)

    c = jnp.linspace(-1.0, 1.0, Q, dtype=jnp.float32)
    c2 = jnp.stack([c, c * c], axis=1)
    wrow = lax.optimization_barrier(
        jnp.broadcast_to(edge_weight[:, None], (E, 16)))

    def step(t, carry):
        f, fu, v, preds, vl = carry
        pin, pout = _sc_edge(fu, v, src, dst, wrow)
        fn, fu2, v2, pred, vls = _tc_call(
            _update_body,
            (_f32((NP, Q)), _f32((NP, 32)), _f32((NP, 16)),
             _f32((N, 2)), _f32((1, 1))),
            f, pin, pout, Wc1, bc1, Wc2, bc2, c2, Wf, bf)
        preds = lax.dynamic_update_slice(preds, pred[None], (t, 0, 0))
        return fn, fu2, v2, preds, vl + vls[0, 0]

    preds0 = jnp.zeros((T_OUT, N, 2), jnp.float32)
    f, fu, v, preds, vl = lax.fori_loop(
        0, T_OUT, step, (f, fu, v, preds0, jnp.float32(0.0)))
    return preds[None], vl / (N * T_OUT)


# speedup vs baseline: 8.6012x; 1.0008x over previous
import functools

import jax
import jax.numpy as jnp
from jax import lax
from jax.experimental import pallas as pl
from jax.experimental.pallas import tpu as pltpu
from jax.experimental.pallas import tpu_sc as plsc

N = 10000
E = 320000
Q = 9
HID = 64
T_OUT = 12
DT = 0.1

NC = 2
NS = 16
NW = NC * NS
EW = E // NW
CH = 80
NCH = EW // CH
GR = CH // 16
NP = 10240
RP = NP // NS
ECH = CH
ENCH = NCH
EGR = ECH // 16
EWP = ECH * ENCH
EP = NW * EWP


def _f32(shape):
    return jax.ShapeDtypeStruct(shape, jnp.float32)


def _mesh():
    return plsc.VectorSubcoreMesh(
        core_axis_name="c", subcore_axis_name="s",
        num_cores=NC, num_subcores=NS)


def _sc_deg_body(dh, out, d_v, ones_v, z_v, acc):
    cid = lax.axis_index("c")
    sid = lax.axis_index("s")
    wid = sid * NC + cid
    row0 = sid * RP
    lanes = lax.iota(jnp.int32, 16)
    zvec = jnp.zeros((16,), jnp.float32)
    evec = jnp.where(lanes == 0, 1.0, 0.0).astype(jnp.float32)

    def fill(i, _):
        ones_v[i] = evec
        return 0

    lax.fori_loop(0, CH, fill, 0)

    def zrow(i, _):
        z_v[i] = zvec
        return 0

    lax.fori_loop(0, RP, zrow, 0)
    pltpu.sync_copy(z_v, acc.at[pl.ds(row0, RP)])
    plsc.subcore_barrier()

    def chunk(ch, _):
        base = wid * EW + ch * CH
        pltpu.sync_copy(dh.at[pl.ds(base, CH)], d_v)
        pltpu.sync_copy(ones_v, acc.at[d_v], add=True)
        return 0

    lax.fori_loop(0, NCH, chunk, 0)
    plsc.subcore_barrier()
    pltpu.sync_copy(acc.at[pl.ds(row0, RP)], out.at[cid, pl.ds(row0, RP)])


def _make_hop_body(D):
    def _hop(th, sh, dh, out, s_v, d_v, rows_v, z_v, acc):
        cid = lax.axis_index("c")
        sid = lax.axis_index("s")
        wid = sid * NC + cid
        row0 = sid * RP
        zvec = jnp.zeros((16,), jnp.float32)

        def zrow(i, _):
            for b in range(D // 16):
                z_v[i, pl.ds(b * 16, 16)] = zvec
            return 0

        lax.fori_loop(0, RP, zrow, 0)
        pltpu.sync_copy(z_v, acc.at[pl.ds(row0, RP)])
        plsc.subcore_barrier()

        def chunk(ch, _):
            base = wid * EW + ch * CH
            pltpu.sync_copy(sh.at[pl.ds(base, CH)], s_v)
            pltpu.sync_copy(dh.at[pl.ds(base, CH)], d_v)
            pltpu.sync_copy(th.at[s_v], rows_v)
            pltpu.sync_copy(rows_v, acc.at[d_v], add=True)
            return 0

        lax.fori_loop(0, NCH, chunk, 0)
        plsc.subcore_barrier()
        pltpu.sync_copy(acc.at[pl.ds(row0, RP)], out.at[cid, pl.ds(row0, RP)])

    return _hop


def _sc_edge_body(fuh, vh, sh, dh, wh, pin, pout,
                  s_v, d_v, si_v, di_v, w_v, fu_v, vv_v, pb_v, z_v,
                  si0, si1, sg0, sg1, ss0, ss1, acc_i, acc_o, fu_s, v_s):
    cid = lax.axis_index("c")
    sid = lax.axis_index("s")
    wid = sid * NC + cid
    row0 = sid * RP
    sidx = (si0, si1)
    sgat = (sg0, sg1)
    ssc = (ss0, ss1)
    zvec = jnp.zeros((16,), jnp.float32)

    pltpu.async_copy(fuh.at[pl.ds(row0, RP)], fu_s.at[pl.ds(row0, RP)], sg0)
    pltpu.async_copy(vh.at[pl.ds(row0, RP)], v_s.at[pl.ds(row0, RP)], sg1)

    def zrow(i, _):
        z_v[i] = zvec
        return 0

    lax.fori_loop(0, RP, zrow, 0)
    pltpu.sync_copy(z_v, acc_i.at[pl.ds(row0, RP)])
    pltpu.sync_copy(z_v, acc_o.at[pl.ds(row0, RP)])
    pltpu.make_async_copy(
        fuh.at[pl.ds(row0, RP)], fu_s.at[pl.ds(row0, RP)], sg0).wait()
    pltpu.make_async_copy(
        vh.at[pl.ds(row0, RP)], v_s.at[pl.ds(row0, RP)], sg1).wait()
    plsc.subcore_barrier()

    def idx_load(c, b):
        base = wid * EWP + c * ECH
        pltpu.async_copy(sh.at[pl.ds(base, ECH)], s_v.at[b], sidx[b])
        pltpu.async_copy(dh.at[pl.ds(base, ECH)], d_v.at[b], sidx[b])
        pltpu.async_copy(wh.at[pl.ds(base, ECH)], w_v.at[b], sidx[b])

    def wait_idx(b):
        pltpu.make_async_copy(sh.at[pl.ds(0, ECH)], s_v.at[b], sidx[b]).wait()
        pltpu.make_async_copy(dh.at[pl.ds(0, ECH)], d_v.at[b], sidx[b]).wait()
        pltpu.make_async_copy(wh.at[pl.ds(0, ECH)], w_v.at[b], sidx[b]).wait()

    def gath(b):
        pltpu.async_copy(fu_s.at[s_v.at[b]], fu_v.at[b], sgat[b])
        pltpu.async_copy(v_s.at[d_v.at[b]], vv_v.at[b], sgat[b])

    def wait_gath(b):
        pltpu.make_async_copy(fu_s.at[s_v.at[b]], fu_v.at[b], sgat[b]).wait()
        pltpu.make_async_copy(v_s.at[d_v.at[b]], vv_v.at[b], sgat[b]).wait()

    def scat(b):
        pltpu.async_copy(pb_v.at[b], acc_i.at[di_v.at[b]], ssc[b], add=True)
        pltpu.async_copy(pb_v.at[b], acc_o.at[si_v.at[b]], ssc[b], add=True)

    def wait_scat(b):
        pltpu.make_async_copy(pb_v.at[b], acc_i.at[di_v.at[b]], ssc[b]).wait()
        pltpu.make_async_copy(pb_v.at[b], acc_o.at[si_v.at[b]], ssc[b]).wait()

    def copy_idx(b):
        for g in range(EGR):
            si_v[b, pl.ds(g * 16, 16)] = s_v[b, pl.ds(g * 16, 16)]
            di_v[b, pl.ds(g * 16, 16)] = d_v[b, pl.ds(g * 16, 16)]

    def compute(b):
        for r in range(ECH):
            fs = fu_v[b, r, pl.ds(0, 16)]
            u = fu_v[b, r, pl.ds(16, 16)]
            z = u + vv_v[b, r]
            pb_v[b, r] = w_v[b, r] * fs / (1.0 + jnp.exp(-z))

    idx_load(0, 0)
    idx_load(1, 1)
    wait_idx(0)
    gath(0)

    def pair(k, _):
        wait_idx(1)
        gath(1)
        wait_gath(0)

        @pl.when(k >= 1)
        def _():
            wait_scat(0)

        copy_idx(0)
        compute(0)
        idx_load(2 * k + 2, 0)
        scat(0)
        wait_idx(0)
        gath(0)
        wait_gath(1)

        @pl.when(k >= 1)
        def _():
            wait_scat(1)

        copy_idx(1)
        compute(1)

        @pl.when(k < (ENCH - 3) // 2)
        def _():
            idx_load(2 * k + 3, 1)

        scat(1)
        return 0

    lax.fori_loop(0, (ENCH - 1) // 2, pair, 0)
    wait_gath(0)
    wait_scat(0)
    copy_idx(0)
    compute(0)
    scat(0)
    wait_scat(0)
    wait_scat(1)
    plsc.subcore_barrier()
    pltpu.sync_copy(acc_i.at[pl.ds(row0, RP)], pin.at[cid, pl.ds(row0, RP)])
    pltpu.sync_copy(acc_o.at[pl.ds(row0, RP)], pout.at[cid, pl.ds(row0, RP)])


@functools.lru_cache(maxsize=1)
def _sc_kernels():
    mesh = _mesh()
    params = pltpu.CompilerParams(
        use_tc_tiling_on_sc=False, needs_layout_passes=False)
    sc_deg = pl.kernel(
        _sc_deg_body,
        out_type=_f32((NC, NP, 16)),
        mesh=mesh,
        compiler_params=params,
        scratch_types=[
            pltpu.VMEM((CH,), jnp.int32),
            pltpu.VMEM((CH, 16), jnp.float32),
            pltpu.VMEM((RP, 16), jnp.float32),
            pltpu.VMEM_SHARED((NP, 16), jnp.float32),
        ],
    )

    def hop(D):
        return pl.kernel(
            _make_hop_body(D),
            out_type=_f32((NC, NP, D)),
            mesh=mesh,
            compiler_params=params,
            scratch_types=[
                pltpu.VMEM((CH,), jnp.int32),
                pltpu.VMEM((CH,), jnp.int32),
                pltpu.VMEM((CH, D), jnp.float32),
                pltpu.VMEM((RP, D), jnp.float32),
                pltpu.VMEM_SHARED((NP, D), jnp.float32),
            ],
        )

    sc_edge = pl.kernel(
        _sc_edge_body,
        out_type=(_f32((NC, NP, 16)), _f32((NC, NP, 16))),
        mesh=mesh,
        compiler_params=params,
        scratch_types=[
            pltpu.VMEM((2, ECH), jnp.int32),
            pltpu.VMEM((2, ECH), jnp.int32),
            pltpu.VMEM((2, ECH), jnp.int32),
            pltpu.VMEM((2, ECH), jnp.int32),
            pltpu.VMEM((2, ECH, 16), jnp.float32),
            pltpu.VMEM((2, ECH, 32), jnp.float32),
            pltpu.VMEM((2, ECH, 16), jnp.float32),
            pltpu.VMEM((2, ECH, 16), jnp.float32),
            pltpu.VMEM((RP, 16), jnp.float32),
            pltpu.SemaphoreType.DMA,
            pltpu.SemaphoreType.DMA,
            pltpu.SemaphoreType.DMA,
            pltpu.SemaphoreType.DMA,
            pltpu.SemaphoreType.DMA,
            pltpu.SemaphoreType.DMA,
            pltpu.VMEM_SHARED((NP, 16), jnp.float32),
            pltpu.VMEM_SHARED((NP, 16), jnp.float32),
            pltpu.VMEM_SHARED((NP, 32), jnp.float32),
            pltpu.VMEM_SHARED((NP, 16), jnp.float32),
        ],
    )
    return sc_deg, hop(16), hop(64), sc_edge


def _tc_call(body, out_shapes, *args):
    return pl.pallas_call(body, out_shape=out_shapes)(*args)


def _norm_body(dp_ref, x_ref, norm_ref, t1_ref):
    deg = dp_ref[0, :, 0:1] + dp_ref[1, :, 0:1]
    norm = lax.rsqrt(jnp.maximum(deg, 1.0))
    norm_ref[...] = norm
    t1_ref[...] = x_ref[...] * norm


def _comb_body(hp_ref, n_ref, h_ref, t_ref):
    norm = n_ref[...]
    h = (hp_ref[0] + hp_ref[1]) * norm
    h_ref[...] = h
    t_ref[...] = h * norm


def _conv1_body(hp_ref, n_ref, x_ref, h1_ref, w_ref, b_ref, out_ref):
    h2 = (hp_ref[0] + hp_ref[1]) * n_ref[...]
    feats = jnp.concatenate(
        [x_ref[:, 0:2], h1_ref[:, 0:2], h2[:, 0:2]], axis=1)
    y = jnp.dot(feats, w_ref[...], preferred_element_type=jnp.float32)
    out_ref[...] = jnp.maximum(y + b_ref[...], 0.0)


def _premul_body(h_ref, n_ref, t_ref):
    t_ref[...] = h_ref[...] * n_ref[...]


def _pack_fuv(f, wf, bf, fu_ref, v_ref):
    u = jnp.dot(f, wf[0:Q], preferred_element_type=jnp.float32)
    vt = jnp.dot(f, wf[Q:2 * Q], preferred_element_type=jnp.float32) + bf
    z7 = jnp.zeros_like(f[:, 0:16 - Q])
    fu_ref[...] = jnp.concatenate([f, z7, u, z7], axis=1)
    v_ref[...] = jnp.concatenate([vt, z7], axis=1)


def _conv2_body(gp_ref, n_ref, h_ref, g1_ref, w_ref, b_ref, wf_ref, bf_ref,
                out_ref, fu_ref, v_ref):
    g2 = (gp_ref[0] + gp_ref[1]) * n_ref[...]
    feats = jnp.concatenate([h_ref[...], g1_ref[...], g2], axis=1)
    y = jnp.dot(feats, w_ref[...], preferred_element_type=jnp.float32) + b_ref[...]
    f = jnp.maximum(y, 0.0) + jnp.log1p(jnp.exp(-jnp.abs(y)))
    out_ref[...] = f
    _pack_fuv(f, wf_ref[...], bf_ref[...], fu_ref, v_ref)


def _update_body(f_ref, pi_ref, po_ref, wc1_ref, bc1_ref, wc2_ref, bc2_ref,
                 c2_ref, wf_ref, bf_ref, fn_ref, fu_ref, v_ref, pred_ref,
                 vl_ref):
    f = f_ref[...]
    infl = pi_ref[0, :, 0:Q] + pi_ref[1, :, 0:Q]
    outf = po_ref[0, :, 0:Q] + po_ref[1, :, 0:Q]
    hid = jnp.maximum(
        jnp.dot(f, wc1_ref[...], preferred_element_type=jnp.float32)
        + bc1_ref[...], 0.0)
    coll = jnp.dot(hid, wc2_ref[...], preferred_element_type=jnp.float32) \
        + bc2_ref[...]
    fn = jnp.maximum(f + DT * (infl - outf + coll), 0.0)
    fn_ref[...] = fn
    _pack_fuv(fn, wf_ref[...], bf_ref[...], fu_ref, v_ref)
    fnv = fn[0:N]
    dens = jnp.sum(fnv, axis=1, keepdims=True)
    moms = jnp.dot(fnv, c2_ref[...], preferred_element_type=jnp.float32)
    vel = moms[:, 0:1] / (dens + 1e-6)
    e2 = moms[:, 1:2] / (dens + 1e-6)
    pred_ref[...] = jnp.concatenate([dens, vel], axis=1)
    vl_ref[...] = jnp.sum(e2 - vel * vel).reshape(1, 1)


def kernel(inputs, edge_index, edge_weight, W1, b1, W2, b2,
           Wc1, bc1, Wc2, bc2, Wf, bf):
    _sc_deg, _sc_hop16, _sc_hop64, _sc_edge = _sc_kernels()
    src = edge_index[0]
    dst = edge_index[1]
    x = inputs[0, -1]
    xpad = jnp.pad(x, ((0, NP - N), (0, 14)))

    dp = _sc_deg(dst)
    norm, t1 = _tc_call(_norm_body, (_f32((NP, 1)), _f32((NP, 16))), dp, xpad)
    hp1 = _sc_hop16(t1, src, dst)
    h1, t2 = _tc_call(_comb_body, (_f32((NP, 16)), _f32((NP, 16))), hp1, norm)
    hp2 = _sc_hop16(t2, src, dst)
    h = _tc_call(_conv1_body, _f32((NP, HID)), hp2, norm, xpad, h1, W1, b1)
    t3 = _tc_call(_premul_body, _f32((NP, HID)), h, norm)
    gp1 = _sc_hop64(t3, src, dst)
    g1, t4 = _tc_call(_comb_body, (_f32((NP, HID)), _f32((NP, HID))), gp1, norm)
    gp2 = _sc_hop64(t4, src, dst)
    f, fu, v = _tc_call(
        _conv2_body, (_f32((NP, Q)), _f32((NP, 32)), _f32((NP, 16))),
        gp2, norm, h, g1, W2, b2, Wf, bf)

    c = jnp.linspace(-1.0, 1.0, Q, dtype=jnp.float32)
    c2 = jnp.stack([c, c * c], axis=1)
    pad = EP - E
    srcp = jnp.concatenate([src, jnp.full((pad,), N, src.dtype)])
    dstp = jnp.concatenate([dst, jnp.full((pad,), N, dst.dtype)])
    wp = jnp.concatenate([edge_weight, jnp.zeros((pad,), jnp.float32)])
    srcp, dstp, wrow = lax.optimization_barrier(
        (srcp, dstp, jnp.broadcast_to(wp[:, None], (EP, 16))))

    def step(t, carry):
        f, fu, v, preds, vl = carry
        pin, pout = _sc_edge(fu, v, srcp, dstp, wrow)
        fn, fu2, v2, pred, vls = _tc_call(
            _update_body,
            (_f32((NP, Q)), _f32((NP, 32)), _f32((NP, 16)),
             _f32((N, 2)), _f32((1, 1))),
            f, pin, pout, Wc1, bc1, Wc2, bc2, c2, Wf, bf)
        preds = lax.dynamic_update_slice(preds, pred[None], (t, 0, 0))
        return fn, fu2, v2, preds, vl + vls[0, 0]

    preds0 = jnp.zeros((T_OUT, N, 2), jnp.float32)
    f, fu, v, preds, vl = lax.fori_loop(
        0, T_OUT, step, (f, fu, v, preds0, jnp.float32(0.0)))
    return preds[None], vl / (N * T_OUT)
